# baseline (speedup 1.0000x reference)
import dataclasses
import functools

import jax
import jax.numpy as jnp
from jax import lax
from jax.experimental import pallas as pl
from jax.experimental.pallas import tpu as pltpu
from jax.experimental.pallas import tpu_sc as plsc


LANES = 16


def _repack_table(table_t, V, E):
    info = plsc.get_sparse_core_info()
    NW = info.num_cores * info.num_subcores
    CHUNK = 384
    PAIRS = CHUNK // 2
    NCHUNK = V // CHUNK
    TAILV = V - NCHUNK * CHUNK
    TPAIRS = TAILV // 2
    NK = -(-NCHUNK // NW)
    NK += NK % 2

    mesh = plsc.VectorSubcoreMesh(core_axis_name="c", subcore_axis_name="s")
    cp = pltpu.CompilerParams()
    if "needs_layout_passes" in pltpu.CompilerParams.__dataclass_fields__:
        cp = dataclasses.replace(cp, needs_layout_passes=False)
    if "use_tc_tiling_on_sc" in pltpu.CompilerParams.__dataclass_fields__:
        cp = dataclasses.replace(cp, use_tc_tiling_on_sc=True)

    @functools.partial(
        pl.kernel,
        out_type=jax.ShapeDtypeStruct((V // 2, 2 * E), jnp.float32),
        mesh=mesh,
        compiler_params=cp,
        scratch_types=[
            pltpu.VMEM((E, CHUNK), jnp.float32),
            pltpu.VMEM((E, CHUNK), jnp.float32),
            pltpu.VMEM((PAIRS, 2 * E), jnp.float32),
            pltpu.VMEM((PAIRS, 2 * E), jnp.float32),
            pltpu.VMEM((E, TAILV), jnp.float32),
            pltpu.VMEM((TPAIRS, 2 * E), jnp.float32),
            pltpu.SemaphoreType.DMA,
            pltpu.SemaphoreType.DMA,
            pltpu.SemaphoreType.DMA,
            pltpu.SemaphoreType.DMA,
        ],
    )
    def run(tt_hbm, out_hbm, in0, in1, st0, st1, tin, tst,
            si0, si1, so0, so1):
        wid = lax.axis_index("s") * info.num_cores + lax.axis_index("c")

        def load(c, buf, sem):
            pltpu.async_copy(tt_hbm.at[:, pl.ds(c * CHUNK, CHUNK)], buf, sem)

        def wait_in(buf, sem):
            pltpu.make_async_copy(tt_hbm.at[:, pl.ds(0, CHUNK)], buf,
                                  sem).wait()

        def store(c, st, sem):
            pltpu.async_copy(st, out_hbm.at[pl.ds(c * PAIRS, PAIRS)], sem)

        def wait_out(st, sem):
            pltpu.make_async_copy(st, out_hbm.at[pl.ds(0, PAIRS)], sem).wait()

        lane = lax.iota(jnp.int32, LANES)
        rowidx = [lane + 16 * u for u in range(E // LANES)]

        def repack(buf, st, npairs=PAIRS):
            @pl.loop(0, npairs)
            def _(p):
                c0 = jnp.full((LANES,), 2 * p, jnp.int32)
                c1 = c0 + 1
                for u in range(E // LANES):
                    st[p, pl.ds(16 * u, LANES)] = plsc.load_gather(
                        buf, [rowidx[u], c0])
                    st[p, pl.ds(E + 16 * u, LANES)] = plsc.load_gather(
                        buf, [rowidx[u], c1])

        load(wid, in0, si0)

        @pl.loop(0, NK // 2)
        def _(t):
            c0 = wid + NW * 2 * t
            c1 = c0 + NW

            @pl.when(c1 < NCHUNK)
            def _():
                load(c1, in1, si1)

            @pl.when(c0 < NCHUNK)
            def _():
                wait_in(in0, si0)

                @pl.when(t > 0)
                def _():
                    wait_out(st0, so0)

                repack(in0, st0)
                store(c0, st0, so0)

            @pl.when(c0 + 2 * NW < NCHUNK)
            def _():
                load(c0 + 2 * NW, in0, si0)

            @pl.when(c1 < NCHUNK)
            def _():
                wait_in(in1, si1)

                @pl.when(t > 0)
                def _():
                    wait_out(st1, so1)

                repack(in1, st1)
                store(c1, st1, so1)

        if TAILV:
            @pl.when(wid == 0)
            def _():
                pltpu.sync_copy(tt_hbm.at[:, pl.ds(NCHUNK * CHUNK, TAILV)],
                                tin)
                repack(tin, tst, npairs=TPAIRS)
                pltpu.sync_copy(tst,
                                out_hbm.at[pl.ds(NCHUNK * PAIRS, TPAIRS)])

        wait_out(st0, so0)
        wait_out(st1, so1)

    return run(table_t)


def _gather_pool(x, x_mask, tbl, B, L, V, E):
    info = plsc.get_sparse_core_info()
    NC, NS = info.num_cores, info.num_subcores
    NW = NC * NS
    assert B % (2 * NW) == 0
    rows_per = B // NW
    H0 = min(128, (L // 2 + 7) // 8 * 8)
    H1 = L - H0
    assert 0 < H1 <= 128 and H0 % 8 == 0 and H1 % 8 == 0
    EV = E // LANES
    MCH = L // LANES
    TAIL = L - MCH * LANES

    mesh = plsc.VectorSubcoreMesh(core_axis_name="c", subcore_axis_name="s")
    cp = pltpu.CompilerParams()
    if "needs_layout_passes" in pltpu.CompilerParams.__dataclass_fields__:
        cp = dataclasses.replace(cp, needs_layout_passes=False)
    if "use_tc_tiling_on_sc" in pltpu.CompilerParams.__dataclass_fields__:
        cp = dataclasses.replace(cp, use_tc_tiling_on_sc=False)

    @functools.partial(
        pl.kernel,
        out_type=jax.ShapeDtypeStruct((B, E), jnp.float32),
        mesh=mesh,
        compiler_params=cp,
        scratch_types=[
            pltpu.VMEM((rows_per, L), jnp.int32),
            pltpu.VMEM((rows_per, L), jnp.float32),
            pltpu.VMEM((L, E), jnp.float32),
            pltpu.VMEM((L, E), jnp.float32),
            pltpu.VMEM((rows_per, E), jnp.float32),
            pltpu.SemaphoreType.DMA,
            pltpu.SemaphoreType.DMA,
        ],
    )
    def run(x_hbm, xm_hbm, tbl_hbm, out_hbm,
            idx_v, mask_v, buf0, buf1, out_stage, sem0, sem1):
        wid = lax.axis_index("s") * NC + lax.axis_index("c")
        base = wid * rows_per
        pltpu.sync_copy(x_hbm.at[pl.ds(base, rows_per)], idx_v)
        pltpu.sync_copy(xm_hbm.at[pl.ds(base, rows_per)], mask_v)

        def issue(row, buf, sem):
            pltpu.async_copy(tbl_hbm.at[idx_v.at[row, pl.ds(0, H0)]],
                             buf.at[pl.ds(0, H0)], sem)
            pltpu.async_copy(tbl_hbm.at[idx_v.at[row, pl.ds(H0, H1)]],
                             buf.at[pl.ds(H0, H1)], sem)

        def drain(buf, sem):
            pltpu.make_async_copy(tbl_hbm.at[pl.ds(0, L)], buf, sem).wait()

        lane_ge_tail = lax.iota(jnp.int32, LANES) >= (LANES - TAIL)

        def inv_num(row):
            msum = jnp.zeros((LANES,), jnp.float32)
            for j in range(MCH):
                msum = msum + mask_v[row, pl.ds(j * LANES, LANES)]
            tail = mask_v[row, pl.ds(L - LANES, LANES)]
            msum = msum + jnp.where(lane_ge_tail, tail, 0.0)
            num = jnp.sum(msum)
            return 1.0 / (jnp.full((LANES,), num, jnp.float32) + 1e-20)

        def reduce_store(row, buf, inv):
            def body(j, accs):
                a0, a1 = accs
                r0 = tuple(
                    a0[k] + (buf[4 * j, pl.ds(k * LANES, LANES)]
                             + buf[4 * j + 1, pl.ds(k * LANES, LANES)])
                    for k in range(EV))
                r1 = tuple(
                    a1[k] + (buf[4 * j + 2, pl.ds(k * LANES, LANES)]
                             + buf[4 * j + 3, pl.ds(k * LANES, LANES)])
                    for k in range(EV))
                return (r0, r1)

            zero = tuple(jnp.zeros((LANES,), jnp.float32) for _ in range(EV))
            a0, a1 = lax.fori_loop(0, L // 4, body, (zero, zero))
            for k in range(EV):
                out_stage[row, pl.ds(k * LANES, LANES)] = (a0[k] + a1[k]) * inv

        issue(0, buf0, sem0)

        @pl.loop(0, rows_per // 2)
        def _(t):
            r0 = 2 * t
            issue(r0 + 1, buf1, sem1)
            inv0 = inv_num(r0)
            drain(buf0, sem0)
            reduce_store(r0, buf0, inv0)

            @pl.when(r0 + 2 < rows_per)
            def _():
                issue(r0 + 2, buf0, sem0)

            inv1 = inv_num(r0 + 1)
            drain(buf1, sem1)
            reduce_store(r0 + 1, buf1, inv1)

        pltpu.sync_copy(out_stage, out_hbm.at[pl.ds(base, rows_per)])

    return run(x, x_mask, tbl)


def kernel(x, x_mask, table):
    B, L = x.shape
    V, E = table.shape
    assert E % LANES == 0 and L % 4 == 0 and V % 2 == 0
    packed = _repack_table(table.T, V, E)
    tbl = packed.reshape(V, E)
    return _gather_pool(x, x_mask, tbl, B, L, V, E)


# speedup vs baseline: 1.9247x; 1.9247x over previous
import dataclasses
import functools

import jax
import jax.numpy as jnp
from jax import lax
from jax.experimental import pallas as pl
from jax.experimental.pallas import tpu as pltpu
from jax.experimental.pallas import tpu_sc as plsc


LANES = 16
CLOG = 13
C = 1 << CLOG
HALF = C // 2


def _repack_table_tc(table_t, V, E):
    NB = -(-V // C)

    TW = 256

    def body(in_ref, out_ref):
        for h in range(2):
            for cc in range(HALF // TW):
                blk = in_ref[:, pl.ds(h * HALF + cc * TW, TW)]
                out_ref[pl.ds(cc * TW, TW), pl.ds(h * E, E)] = blk[...].T

    return pl.pallas_call(
        body,
        grid=(NB,),
        in_specs=[pl.BlockSpec((E, C), lambda i: (0, i))],
        out_specs=pl.BlockSpec((HALF, 2 * E), lambda i: (i, 0)),
        out_shape=jax.ShapeDtypeStruct((NB * HALF, 2 * E), jnp.float32),
    )(table_t)


def _gather_pool(x, x_mask, packed, B, L, E):
    info = plsc.get_sparse_core_info()
    NC, NS = info.num_cores, info.num_subcores
    NW = NC * NS
    assert B % (2 * NW) == 0
    rows_per = B // NW
    H0 = min(128, L)
    H1 = L - H0
    assert 0 < H1 <= 128 and H0 % 8 == 0 and H1 % 8 == 0
    EV = E // LANES
    MCH = L // LANES
    TAIL = L - MCH * LANES

    mesh = plsc.VectorSubcoreMesh(core_axis_name="c", subcore_axis_name="s")
    cp = pltpu.CompilerParams()
    if "needs_layout_passes" in pltpu.CompilerParams.__dataclass_fields__:
        cp = dataclasses.replace(cp, needs_layout_passes=False)
    if "use_tc_tiling_on_sc" in pltpu.CompilerParams.__dataclass_fields__:
        cp = dataclasses.replace(cp, use_tc_tiling_on_sc=True)

    @functools.partial(
        pl.kernel,
        out_type=jax.ShapeDtypeStruct((B // 2, 2 * E), jnp.float32),
        mesh=mesh,
        compiler_params=cp,
        scratch_types=[
            pltpu.VMEM((rows_per // 2, L), jnp.int32),
            pltpu.VMEM((rows_per, L), jnp.float32),
            pltpu.VMEM((L, 2 * E), jnp.float32),
            pltpu.VMEM((L, 2 * E), jnp.float32),
            pltpu.VMEM((2, 128), jnp.int32),
            pltpu.VMEM((2, 128), jnp.int32),
            pltpu.VMEM((MCH + 1, LANES), jnp.float32),
            pltpu.VMEM((MCH + 1, LANES), jnp.float32),
            pltpu.VMEM((rows_per // 2, 2 * E), jnp.float32),
            pltpu.SemaphoreType.DMA,
            pltpu.SemaphoreType.DMA,
        ],
    )
    def run(x_hbm, xm_hbm, tbl_hbm, out_hbm,
            idx_v, mask_v, buf0, buf1, pidx0, pidx1, sel0, sel1,
            out_stage, sem0, sem1):
        wid = lax.axis_index("s") * NC + lax.axis_index("c")
        base = wid * rows_per
        HR = rows_per // 2
        pltpu.sync_copy(x_hbm.at[pl.ds(base, HR)], idx_v)
        pltpu.sync_copy(xm_hbm.at[pl.ds(base, rows_per)], mask_v)

        def prep_idx(row, pidx, sel):
            lrow = row & (HR - 1)
            for j in range(MCH + 1):
                src = min(j * LANES, L - LANES)
                v = idx_v[lrow, pl.ds(src, LANES)]
                pr = ((v >> CLOG) << (CLOG - 1)) | (v & (HALF - 1))
                r, c = divmod(src, 128)
                pidx[r, pl.ds(c, LANES)] = pr
                sel[j, pl.ds(0, LANES)] = (
                    (v >> (CLOG - 1)) & 1).astype(jnp.float32)

        def issue(row, pidx, sel, buf, sem):
            prep_idx(row, pidx, sel)
            pltpu.async_copy(tbl_hbm.at[pidx.at[0]],
                             buf.at[pl.ds(0, H0)], sem)
            pltpu.async_copy(tbl_hbm.at[pidx.at[1, pl.ds(0, H1)]],
                             buf.at[pl.ds(H0, H1)], sem)

        def drain(buf, sem):
            pltpu.make_async_copy(tbl_hbm.at[pl.ds(0, L)], buf, sem).wait()

        lane_ge_tail = lax.iota(jnp.int32, LANES) >= (LANES - TAIL)

        def inv_num(row):
            msum = jnp.zeros((LANES,), jnp.float32)
            for j in range(MCH):
                msum = msum + mask_v[row, pl.ds(j * LANES, LANES)]
            tail = mask_v[row, pl.ds(L - LANES, LANES)]
            msum = msum + jnp.where(lane_ge_tail, tail, 0.0)
            num = jnp.sum(msum)
            return 1.0 / (jnp.full((LANES,), num, jnp.float32) + 1e-20)

        def reduce_store(row, buf, sel, inv):
            def chunk_body(sel16, lo_r, s_range, accs):
                a0, a1 = accs
                for s in s_range:
                    j = lo_r + s
                    ps = jnp.full((LANES,), sel16[s], jnp.float32)
                    upd = []
                    acc = a0 if s % 2 == 0 else a1
                    for k in range(EV):
                        lo = buf[j, pl.ds(k * LANES, LANES)]
                        hi = buf[j, pl.ds(E + k * LANES, LANES)]
                        upd.append(acc[k] + (lo + ps * (hi - lo)))
                    if s % 2 == 0:
                        a0 = tuple(upd)
                    else:
                        a1 = tuple(upd)
                return (a0, a1)

            def body(c, accs):
                return chunk_body(sel[c, pl.ds(0, LANES)], c * LANES,
                                  range(LANES), accs)

            zero = tuple(jnp.zeros((LANES,), jnp.float32) for _ in range(EV))
            accs = lax.fori_loop(0, MCH, body, (zero, zero))
            stail = sel[MCH, pl.ds(0, LANES)]
            a0, a1 = chunk_body(stail, L - LANES,
                                range(LANES - TAIL, LANES), accs)
            col = (row % 2) * E
            for k in range(EV):
                out_stage[row // 2, pl.ds(col + k * LANES, LANES)] = (
                    (a0[k] + a1[k]) * inv)

        issue(0, pidx0, sel0, buf0, sem0)

        @pl.loop(0, rows_per // 2)
        def _(t):
            r0 = 2 * t
            issue(r0 + 1, pidx1, sel1, buf1, sem1)
            inv0 = inv_num(r0)
            drain(buf0, sem0)
            reduce_store(r0, buf0, sel0, inv0)

            @pl.when(r0 + 2 == HR)
            def _():
                pltpu.sync_copy(x_hbm.at[pl.ds(base + HR, HR)], idx_v)

            @pl.when(r0 + 2 < rows_per)
            def _():
                issue(r0 + 2, pidx0, sel0, buf0, sem0)

            inv1 = inv_num(r0 + 1)
            drain(buf1, sem1)
            reduce_store(r0 + 1, buf1, sel1, inv1)

        pltpu.sync_copy(out_stage,
                        out_hbm.at[pl.ds(wid * (rows_per // 2),
                                         rows_per // 2)])

    return run(x, x_mask, packed)


def kernel(x, x_mask, table):
    B, L = x.shape
    V, E = table.shape
    assert E % LANES == 0 and L % 4 == 0
    packed = _repack_table_tc(table.T, V, E)
    return _gather_pool(x, x_mask, packed, B, L, E).reshape(B, E)


# speedup vs baseline: 1.9344x; 1.0051x over previous
import dataclasses
import functools

import jax
import jax.numpy as jnp
from jax import lax
from jax.experimental import pallas as pl
from jax.experimental.pallas import tpu as pltpu
from jax.experimental.pallas import tpu_sc as plsc


LANES = 16
CLOG = 13
C = 1 << CLOG
HALF = C // 2


def _repack_table_tc(table_t, V, E):
    NB = -(-V // C)

    TW = 256

    def body(in_ref, out_ref):
        for h in range(2):
            for cc in range(HALF // TW):
                blk = in_ref[:, pl.ds(h * HALF + cc * TW, TW)]
                out_ref[pl.ds(cc * TW, TW), pl.ds(h * E, E)] = blk[...].T

    return pl.pallas_call(
        body,
        grid=(NB,),
        in_specs=[pl.BlockSpec((E, C), lambda i: (0, i))],
        out_specs=pl.BlockSpec((HALF, 2 * E), lambda i: (i, 0)),
        out_shape=jax.ShapeDtypeStruct((NB * HALF, 2 * E), jnp.float32),
    )(table_t)


def _gather_pool(x, x_mask, packed, B, L, E):
    info = plsc.get_sparse_core_info()
    NC, NS = info.num_cores, info.num_subcores
    NW = NC * NS
    assert B % (2 * NW) == 0
    rows_per = B // NW
    H0 = min(128, L)
    H1 = L - H0
    assert 0 < H1 <= 128 and H0 % 8 == 0 and H1 % 8 == 0
    EV = E // LANES
    MCH = L // LANES
    TAIL = L - MCH * LANES

    mesh = plsc.VectorSubcoreMesh(core_axis_name="c", subcore_axis_name="s")
    cp = pltpu.CompilerParams()
    if "needs_layout_passes" in pltpu.CompilerParams.__dataclass_fields__:
        cp = dataclasses.replace(cp, needs_layout_passes=False)
    if "use_tc_tiling_on_sc" in pltpu.CompilerParams.__dataclass_fields__:
        cp = dataclasses.replace(cp, use_tc_tiling_on_sc=True)

    @functools.partial(
        pl.kernel,
        out_type=jax.ShapeDtypeStruct((B // 2, 2 * E), jnp.float32),
        mesh=mesh,
        compiler_params=cp,
        scratch_types=[
            pltpu.VMEM((rows_per // 2, L), jnp.int32),
            pltpu.VMEM((rows_per, L), jnp.float32),
            pltpu.VMEM((L, 2 * E), jnp.float32),
            pltpu.VMEM((L, 2 * E), jnp.float32),
            pltpu.VMEM((2, 128), jnp.int32),
            pltpu.VMEM((2, 128), jnp.int32),
            pltpu.VMEM((MCH + 1, LANES), jnp.float32),
            pltpu.VMEM((MCH + 1, LANES), jnp.float32),
            pltpu.VMEM((rows_per // 2, 2 * E), jnp.float32),
            pltpu.SemaphoreType.DMA,
            pltpu.SemaphoreType.DMA,
        ],
    )
    def run(x_hbm, xm_hbm, tbl_hbm, out_hbm,
            idx_v, mask_v, buf0, buf1, pidx0, pidx1, sel0, sel1,
            out_stage, sem0, sem1):
        wid = lax.axis_index("s") * NC + lax.axis_index("c")
        base = wid * rows_per
        HR = rows_per // 2
        pltpu.sync_copy(x_hbm.at[pl.ds(base, HR)], idx_v)
        pltpu.sync_copy(xm_hbm.at[pl.ds(base, rows_per)], mask_v)

        def prep_idx(row, pidx, sel):
            lrow = row & (HR - 1)
            for j in range(MCH + 1):
                src = min(j * LANES, L - LANES)
                v = idx_v[lrow, pl.ds(src, LANES)]
                pr = ((v >> CLOG) << (CLOG - 1)) | (v & (HALF - 1))
                r, c = divmod(src, 128)
                pidx[r, pl.ds(c, LANES)] = pr
                sel[j, pl.ds(0, LANES)] = (
                    (v >> (CLOG - 1)) & 1).astype(jnp.float32)

        def issue(row, pidx, sel, buf, sem):
            prep_idx(row, pidx, sel)
            pltpu.async_copy(tbl_hbm.at[pidx.at[0]],
                             buf.at[pl.ds(0, H0)], sem)
            pltpu.async_copy(tbl_hbm.at[pidx.at[1, pl.ds(0, H1)]],
                             buf.at[pl.ds(H0, H1)], sem)

        def drain(buf, sem):
            pltpu.make_async_copy(tbl_hbm.at[pl.ds(0, L)], buf, sem).wait()

        lane_ge_tail = lax.iota(jnp.int32, LANES) >= (LANES - TAIL)

        def inv_num(row):
            msum = jnp.zeros((LANES,), jnp.float32)
            for j in range(MCH):
                msum = msum + mask_v[row, pl.ds(j * LANES, LANES)]
            tail = mask_v[row, pl.ds(L - LANES, LANES)]
            msum = msum + jnp.where(lane_ge_tail, tail, 0.0)
            num = jnp.sum(msum)
            return 1.0 / (jnp.full((LANES,), num, jnp.float32) + 1e-20)

        def reduce_store(row, buf, sel, inv):
            def chunk_body(sel16, lo_r, s_range, accs):
                a0, a1 = accs
                for s in s_range:
                    j = lo_r + s
                    ps = lax.gather(
                        sel16,
                        jnp.full((LANES, 1), s, jnp.int32),
                        lax.GatherDimensionNumbers(
                            offset_dims=(), collapsed_slice_dims=(0,),
                            start_index_map=(0,)),
                        (1,),
                        mode=lax.GatherScatterMode.PROMISE_IN_BOUNDS)
                    upd = []
                    acc = a0 if s % 2 == 0 else a1
                    for k in range(EV):
                        lo = buf[j, pl.ds(k * LANES, LANES)]
                        hi = buf[j, pl.ds(E + k * LANES, LANES)]
                        upd.append(acc[k] + (lo + ps * (hi - lo)))
                    if s % 2 == 0:
                        a0 = tuple(upd)
                    else:
                        a1 = tuple(upd)
                return (a0, a1)

            def body(c, accs):
                return chunk_body(sel[c, pl.ds(0, LANES)], c * LANES,
                                  range(LANES), accs)

            zero = tuple(jnp.zeros((LANES,), jnp.float32) for _ in range(EV))
            accs = lax.fori_loop(0, MCH, body, (zero, zero))
            stail = sel[MCH, pl.ds(0, LANES)]
            a0, a1 = chunk_body(stail, L - LANES,
                                range(LANES - TAIL, LANES), accs)
            col = (row % 2) * E
            for k in range(EV):
                out_stage[row // 2, pl.ds(col + k * LANES, LANES)] = (
                    (a0[k] + a1[k]) * inv)

        issue(0, pidx0, sel0, buf0, sem0)

        @pl.loop(0, rows_per // 2)
        def _(t):
            r0 = 2 * t
            issue(r0 + 1, pidx1, sel1, buf1, sem1)
            inv0 = inv_num(r0)
            drain(buf0, sem0)
            reduce_store(r0, buf0, sel0, inv0)

            @pl.when(r0 + 2 == HR)
            def _():
                pltpu.sync_copy(x_hbm.at[pl.ds(base + HR, HR)], idx_v)

            @pl.when(r0 + 2 < rows_per)
            def _():
                issue(r0 + 2, pidx0, sel0, buf0, sem0)

            inv1 = inv_num(r0 + 1)
            drain(buf1, sem1)
            reduce_store(r0 + 1, buf1, sel1, inv1)

        pltpu.sync_copy(out_stage,
                        out_hbm.at[pl.ds(wid * (rows_per // 2),
                                         rows_per // 2)])

    return run(x, x_mask, packed)


def kernel(x, x_mask, table):
    B, L = x.shape
    V, E = table.shape
    assert E % LANES == 0 and L % 4 == 0
    packed = _repack_table_tc(table.T, V, E)
    return _gather_pool(x, x_mask, packed, B, L, E).reshape(B, E)


# speedup vs baseline: 2.2491x; 1.1627x over previous
import dataclasses
import functools

import jax
import jax.numpy as jnp
from jax import lax
from jax.experimental import pallas as pl
from jax.experimental.pallas import tpu as pltpu
from jax.experimental.pallas import tpu_sc as plsc


LANES = 16


def kernel(x, x_mask, table):
    B, L = x.shape
    V, E = table.shape
    info = plsc.get_sparse_core_info()
    NC, NS = info.num_cores, info.num_subcores
    NW = NC * NS
    assert B % (2 * NW) == 0
    rows_per = B // NW
    assert L % 4 == 0
    H0 = min(128, (L // 2 + 7) // 8 * 8)
    H1 = L - H0
    assert 0 < H1 <= 128 and H0 % 8 == 0 and H1 % 8 == 0
    EV = E // LANES
    assert E % LANES == 0
    MCH = L // LANES
    TAIL = L - MCH * LANES

    mesh = plsc.VectorSubcoreMesh(core_axis_name="c", subcore_axis_name="s")
    cp = pltpu.CompilerParams()
    if "needs_layout_passes" in pltpu.CompilerParams.__dataclass_fields__:
        cp = dataclasses.replace(cp, needs_layout_passes=False)
    if "use_tc_tiling_on_sc" in pltpu.CompilerParams.__dataclass_fields__:
        cp = dataclasses.replace(cp, use_tc_tiling_on_sc=False)

    @functools.partial(
        pl.kernel,
        out_type=jax.ShapeDtypeStruct((B, E), jnp.float32),
        mesh=mesh,
        compiler_params=cp,
        scratch_types=[
            pltpu.VMEM((rows_per, L), jnp.int32),
            pltpu.VMEM((rows_per, L), jnp.float32),
            pltpu.VMEM((L, E), jnp.float32),
            pltpu.VMEM((L, E), jnp.float32),
            pltpu.VMEM((rows_per, E), jnp.float32),
            pltpu.SemaphoreType.DMA,
            pltpu.SemaphoreType.DMA,
        ],
    )
    def run(x_hbm, xm_hbm, tbl_hbm, out_hbm,
            idx_v, mask_v, buf0, buf1, out_stage, sem0, sem1):
        wid = lax.axis_index("s") * NC + lax.axis_index("c")
        base = wid * rows_per
        pltpu.sync_copy(x_hbm.at[pl.ds(base, rows_per)], idx_v)
        pltpu.sync_copy(xm_hbm.at[pl.ds(base, rows_per)], mask_v)

        def issue(row, buf, sem):
            pltpu.async_copy(tbl_hbm.at[idx_v.at[row, pl.ds(0, H0)]],
                             buf.at[pl.ds(0, H0)], sem)
            pltpu.async_copy(tbl_hbm.at[idx_v.at[row, pl.ds(H0, H1)]],
                             buf.at[pl.ds(H0, H1)], sem)

        def drain(buf, sem):
            pltpu.make_async_copy(tbl_hbm.at[pl.ds(0, L)], buf, sem).wait()

        lane_ge_tail = lax.iota(jnp.int32, LANES) >= (LANES - TAIL)

        def inv_num(row):
            msum = jnp.zeros((LANES,), jnp.float32)
            for j in range(MCH):
                msum = msum + mask_v[row, pl.ds(j * LANES, LANES)]
            tail = mask_v[row, pl.ds(L - LANES, LANES)]
            msum = msum + jnp.where(lane_ge_tail, tail, 0.0)
            num = jnp.sum(msum)
            return 1.0 / (jnp.full((LANES,), num, jnp.float32) + 1e-20)

        def reduce_store(row, buf, inv):
            def body(j, accs):
                a0, a1 = accs
                r0 = tuple(
                    a0[k] + (buf[4 * j, pl.ds(k * LANES, LANES)]
                             + buf[4 * j + 1, pl.ds(k * LANES, LANES)])
                    for k in range(EV))
                r1 = tuple(
                    a1[k] + (buf[4 * j + 2, pl.ds(k * LANES, LANES)]
                             + buf[4 * j + 3, pl.ds(k * LANES, LANES)])
                    for k in range(EV))
                return (r0, r1)

            zero = tuple(jnp.zeros((LANES,), jnp.float32) for _ in range(EV))
            a0, a1 = lax.fori_loop(0, L // 4, body, (zero, zero))
            for k in range(EV):
                out_stage[row, pl.ds(k * LANES, LANES)] = (a0[k] + a1[k]) * inv

        issue(0, buf0, sem0)

        @pl.loop(0, rows_per // 2)
        def _(t):
            r0 = 2 * t
            issue(r0 + 1, buf1, sem1)
            inv0 = inv_num(r0)
            drain(buf0, sem0)
            reduce_store(r0, buf0, inv0)

            @pl.when(r0 + 2 < rows_per)
            def _():
                issue(r0 + 2, buf0, sem0)

            inv1 = inv_num(r0 + 1)
            drain(buf1, sem1)
            reduce_store(r0 + 1, buf1, inv1)

        pltpu.sync_copy(out_stage, out_hbm.at[pl.ds(base, rows_per)])

    return run(x, x_mask, table)
